# baseline (speedup 1.0000x reference)
import functools

import jax
import jax.numpy as jnp
from jax import lax
from jax.experimental import pallas as pl
from jax.experimental.pallas import tpu as pltpu
from jax.experimental.pallas import tpu_sc as plsc

N_NODES = 10000
D = 128
N_CORES = 2
N_SUBCORES = 16
N_WORKERS = N_CORES * N_SUBCORES
CHUNK = 64
ACC_ROWS = 10240
FRAC0 = 0.55


def _matmul_kernel(x_ref, w_ref, o_ref):
    o_ref[...] = jnp.dot(x_ref[...], w_ref[...],
                         preferred_element_type=jnp.float32)


def _combine_kernel(p0_ref, p1_ref, o_ref):
    o_ref[...] = jnp.maximum(p0_ref[0] + p1_ref[0], 0.0)


def _sc_agg_body(x_hbm, pk_hbm, w_hbm, part_hbm,
                 pk_v, w_v, colbuf, rowbuf, rows2, acc_sh, sem_a, sem_b,
                 nc0, nc1):
    cid = lax.axis_index("c")
    sid = lax.axis_index("s")
    wid = cid * N_SUBCORES + sid
    n_chunks = jnp.where(cid == 0, nc0, nc1)
    rows_a = rows2.at[pl.ds(0, CHUNK)]
    rows_b = rows2.at[pl.ds(CHUNK, CHUNK)]

    pltpu.sync_copy(pk_hbm.at[wid], pk_v)
    pltpu.sync_copy(w_hbm.at[wid], w_v)

    def _unpack(ch, p):
        for g in range(CHUNK // 16):
            sl = pl.ds(g * 16, 16)
            v = pk_v[ch, sl]
            colbuf[p, sl] = v & 0xFFFF
            rowbuf[p, sl] = lax.shift_right_logical(v, 16)

    zeros16 = jnp.zeros((16,), jnp.float32)

    def _zero_row(i, _):
        for j in range(D // 16):
            rows2[i, pl.ds(j * 16, 16)] = zeros16
        return 0

    lax.fori_loop(0, 2 * CHUNK, _zero_row, 0)

    rows_per_sub = ACC_ROWS // N_SUBCORES
    for k in range(rows_per_sub // (2 * CHUNK)):
        pltpu.sync_copy(
            rows2,
            acc_sh.at[pl.ds(sid * rows_per_sub + k * 2 * CHUNK, 2 * CHUNK)])
    plsc.subcore_barrier()

    def _fire_gather(p, buf, sem):
        pltpu.async_copy(x_hbm.at[colbuf.at[p]], buf, sem)

    def _wait_gather(p, buf, sem):
        pltpu.make_async_copy(x_hbm.at[colbuf.at[p]], buf, sem).wait()

    def _scale(ch, off):
        def _group_body(g, _):
            w16 = w_v[ch, pl.ds(g * 16, 16)]
            base = g * 16 + off
            for l in range(16):
                wvec = jnp.full((16,), w16[l], jnp.float32)
                for j in range(D // 16):
                    sl = pl.ds(j * 16, 16)
                    rows2[base + l, sl] = rows2[base + l, sl] * wvec
            return 0

        lax.fori_loop(0, CHUNK // 16, _group_body, 0)

    _unpack(0, 0)
    _fire_gather(0, rows_a, sem_a)

    def _pair_body(q, _):
        ch0 = 2 * q
        for b, (buf, off, sem, osem, obuf) in enumerate(
                ((rows_a, 0, sem_a, sem_b, rows_b),
                 (rows_b, CHUNK, sem_b, sem_a, rows_a))):
            ch = ch0 + b
            p, op = b, 1 - b
            @pl.when(ch + 1 < n_chunks)
            def _():
                _unpack(ch + 1, op)
                _fire_gather(op, obuf, osem)

            _wait_gather(p, buf, sem)
            _scale(ch, off)
            pltpu.sync_copy(buf, acc_sh.at[rowbuf.at[p]], add=True)
        return 0

    lax.fori_loop(0, n_chunks // 2, _pair_body, 0)
    plsc.subcore_barrier()

    sl = pl.ds(sid * rows_per_sub, rows_per_sub)
    pltpu.sync_copy(acc_sh.at[sl], part_hbm.at[cid, sl])


def kernel(inputs, edge_index, edge_weight, W):
    n_edges = edge_index.shape[1]
    total = -(-n_edges // (N_SUBCORES * CHUNK))
    nc0 = 2 * int(round(FRAC0 * total / 2))
    nc1 = 2 * (-(-(total - nc0) // 2))
    ncmax = max(nc0, nc1)
    cap = N_SUBCORES * CHUNK * (nc0 + nc1)

    def _layout(a):
        a = jnp.pad(a, (0, cap - n_edges))
        s0 = a[:N_SUBCORES * nc0 * CHUNK].reshape(N_SUBCORES, nc0, CHUNK)
        s1 = a[N_SUBCORES * nc0 * CHUNK:].reshape(N_SUBCORES, nc1, CHUNK)
        s0 = jnp.pad(s0, ((0, 0), (0, ncmax - nc0), (0, 0)))
        s1 = jnp.pad(s1, ((0, 0), (0, ncmax - nc1), (0, 0)))
        return jnp.concatenate([s0, s1], axis=0)

    col = edge_index[1].astype(jnp.int32)
    row = edge_index[0].astype(jnp.int32)
    pk_p = _layout((row << 16) | col)
    w_p = _layout(edge_weight)

    n = inputs.shape[0]
    blk = 1000
    x = pl.pallas_call(
        _matmul_kernel,
        grid=(n // blk,),
        in_specs=[
            pl.BlockSpec((blk, D), lambda i: (i, 0)),
            pl.BlockSpec((D, D), lambda i: (0, 0)),
        ],
        out_specs=pl.BlockSpec((blk, D), lambda i: (i, 0)),
        out_shape=jax.ShapeDtypeStruct((n, D), jnp.float32),
    )(inputs, W)

    mesh = plsc.VectorSubcoreMesh(core_axis_name="c", subcore_axis_name="s")
    sc_agg = functools.partial(
        pl.kernel,
        out_type=jax.ShapeDtypeStruct((N_CORES, ACC_ROWS, D), jnp.float32),
        mesh=mesh,
        scratch_types=[
            pltpu.VMEM((ncmax, CHUNK), jnp.int32),
            pltpu.VMEM((ncmax, CHUNK), jnp.float32),
            pltpu.VMEM((2, CHUNK), jnp.int32),
            pltpu.VMEM((2, CHUNK), jnp.int32),
            pltpu.VMEM((2 * CHUNK, D), jnp.float32),
            pltpu.VMEM_SHARED((ACC_ROWS, D), jnp.float32),
            pltpu.SemaphoreType.DMA,
            pltpu.SemaphoreType.DMA,
        ],
        compiler_params=pltpu.CompilerParams(use_tc_tiling_on_sc=False),
    )(functools.partial(_sc_agg_body, nc0=nc0, nc1=nc1))
    part = sc_agg(x, pk_p, w_p)

    out = pl.pallas_call(
        _combine_kernel,
        grid=(n // blk,),
        in_specs=[
            pl.BlockSpec((1, blk, D), lambda i: (0, i, 0)),
            pl.BlockSpec((1, blk, D), lambda i: (1, i, 0)),
        ],
        out_specs=pl.BlockSpec((blk, D), lambda i: (i, 0)),
        out_shape=jax.ShapeDtypeStruct((n, D), jnp.float32),
    )(part, part)
    return out


# speedup vs baseline: 1.0089x; 1.0089x over previous
import functools

import jax
import jax.numpy as jnp
from jax import lax
from jax.experimental import pallas as pl
from jax.experimental.pallas import tpu as pltpu
from jax.experimental.pallas import tpu_sc as plsc

N_NODES = 10000
D = 128
N_CORES = 2
N_SUBCORES = 16
N_WORKERS = N_CORES * N_SUBCORES
CHUNK = 64
ACC_ROWS = 10240
FRAC0 = 0.60


def _matmul_kernel(x_ref, w_ref, o_ref):
    o_ref[...] = jnp.dot(x_ref[...], w_ref[...],
                         preferred_element_type=jnp.float32)


def _combine_kernel(p0_ref, p1_ref, o_ref):
    o_ref[...] = jnp.maximum(p0_ref[0] + p1_ref[0], 0.0)


def _sc_agg_body(x_hbm, pk_hbm, w_hbm, part_hbm,
                 pk_v, w_v, colbuf, rowbuf, rows2, acc_sh, sem_a, sem_b,
                 nc0, nc1):
    cid = lax.axis_index("c")
    sid = lax.axis_index("s")
    wid = cid * N_SUBCORES + sid
    n_chunks = jnp.where(cid == 0, nc0, nc1)
    rows_a = rows2.at[pl.ds(0, CHUNK)]
    rows_b = rows2.at[pl.ds(CHUNK, CHUNK)]

    pltpu.sync_copy(pk_hbm.at[wid], pk_v)
    pltpu.sync_copy(w_hbm.at[wid], w_v)

    def _unpack(ch, p):
        for g in range(CHUNK // 16):
            sl = pl.ds(g * 16, 16)
            v = pk_v[ch, sl]
            colbuf[p, sl] = v & 0xFFFF
            rowbuf[p, sl] = lax.shift_right_logical(v, 16)

    zeros16 = jnp.zeros((16,), jnp.float32)

    def _zero_row(i, _):
        for j in range(D // 16):
            rows2[i, pl.ds(j * 16, 16)] = zeros16
        return 0

    lax.fori_loop(0, 2 * CHUNK, _zero_row, 0)

    rows_per_sub = ACC_ROWS // N_SUBCORES
    for k in range(rows_per_sub // (2 * CHUNK)):
        pltpu.sync_copy(
            rows2,
            acc_sh.at[pl.ds(sid * rows_per_sub + k * 2 * CHUNK, 2 * CHUNK)])
    plsc.subcore_barrier()

    def _fire_gather(p, buf, sem):
        pltpu.async_copy(x_hbm.at[colbuf.at[p]], buf, sem)

    def _wait_gather(p, buf, sem):
        pltpu.make_async_copy(x_hbm.at[colbuf.at[p]], buf, sem).wait()

    def _scale(ch, off):
        def _group_body(g, _):
            w16 = w_v[ch, pl.ds(g * 16, 16)]
            base = g * 16 + off
            for l in range(16):
                wvec = jnp.full((16,), w16[l], jnp.float32)
                for j in range(D // 16):
                    sl = pl.ds(j * 16, 16)
                    rows2[base + l, sl] = rows2[base + l, sl] * wvec
            return 0

        lax.fori_loop(0, CHUNK // 16, _group_body, 0)

    _unpack(0, 0)
    _fire_gather(0, rows_a, sem_a)

    def _pair_body(q, _):
        ch0 = 2 * q
        for b, (buf, off, sem, osem, obuf) in enumerate(
                ((rows_a, 0, sem_a, sem_b, rows_b),
                 (rows_b, CHUNK, sem_b, sem_a, rows_a))):
            ch = ch0 + b
            p, op = b, 1 - b
            @pl.when(ch + 1 < n_chunks)
            def _():
                _unpack(ch + 1, op)
                _fire_gather(op, obuf, osem)

            _wait_gather(p, buf, sem)
            _scale(ch, off)
            pltpu.sync_copy(buf, acc_sh.at[rowbuf.at[p]], add=True)
        return 0

    lax.fori_loop(0, n_chunks // 2, _pair_body, 0)
    plsc.subcore_barrier()

    sl = pl.ds(sid * rows_per_sub, rows_per_sub)
    pltpu.sync_copy(acc_sh.at[sl], part_hbm.at[cid, sl])


def kernel(inputs, edge_index, edge_weight, W):
    n_edges = edge_index.shape[1]
    total = -(-n_edges // (N_SUBCORES * CHUNK))
    nc0 = 2 * int(round(FRAC0 * total / 2))
    nc1 = 2 * (-(-(total - nc0) // 2))
    ncmax = max(nc0, nc1)
    cap = N_SUBCORES * CHUNK * (nc0 + nc1)

    def _layout(a):
        a = jnp.pad(a, (0, cap - n_edges))
        s0 = a[:N_SUBCORES * nc0 * CHUNK].reshape(N_SUBCORES, nc0, CHUNK)
        s1 = a[N_SUBCORES * nc0 * CHUNK:].reshape(N_SUBCORES, nc1, CHUNK)
        s0 = jnp.pad(s0, ((0, 0), (0, ncmax - nc0), (0, 0)))
        s1 = jnp.pad(s1, ((0, 0), (0, ncmax - nc1), (0, 0)))
        return jnp.concatenate([s0, s1], axis=0)

    col = edge_index[1].astype(jnp.int32)
    row = edge_index[0].astype(jnp.int32)
    pk_p = _layout((row << 16) | col)
    w_p = _layout(edge_weight)

    n = inputs.shape[0]
    blk = 1000
    x = pl.pallas_call(
        _matmul_kernel,
        grid=(n // blk,),
        in_specs=[
            pl.BlockSpec((blk, D), lambda i: (i, 0)),
            pl.BlockSpec((D, D), lambda i: (0, 0)),
        ],
        out_specs=pl.BlockSpec((blk, D), lambda i: (i, 0)),
        out_shape=jax.ShapeDtypeStruct((n, D), jnp.float32),
    )(inputs, W)

    mesh = plsc.VectorSubcoreMesh(core_axis_name="c", subcore_axis_name="s")
    sc_agg = functools.partial(
        pl.kernel,
        out_type=jax.ShapeDtypeStruct((N_CORES, ACC_ROWS, D), jnp.float32),
        mesh=mesh,
        scratch_types=[
            pltpu.VMEM((ncmax, CHUNK), jnp.int32),
            pltpu.VMEM((ncmax, CHUNK), jnp.float32),
            pltpu.VMEM((2, CHUNK), jnp.int32),
            pltpu.VMEM((2, CHUNK), jnp.int32),
            pltpu.VMEM((2 * CHUNK, D), jnp.float32),
            pltpu.VMEM_SHARED((ACC_ROWS, D), jnp.float32),
            pltpu.SemaphoreType.DMA,
            pltpu.SemaphoreType.DMA,
        ],
        compiler_params=pltpu.CompilerParams(use_tc_tiling_on_sc=False),
    )(functools.partial(_sc_agg_body, nc0=nc0, nc1=nc1))
    part = sc_agg(x, pk_p, w_p)

    out = pl.pallas_call(
        _combine_kernel,
        grid=(n // blk,),
        in_specs=[
            pl.BlockSpec((1, blk, D), lambda i: (0, i, 0)),
            pl.BlockSpec((1, blk, D), lambda i: (1, i, 0)),
        ],
        out_specs=pl.BlockSpec((blk, D), lambda i: (i, 0)),
        out_shape=jax.ShapeDtypeStruct((n, D), jnp.float32),
    )(part, part)
    return out


# speedup vs baseline: 1.0091x; 1.0002x over previous
import functools

import jax
import jax.numpy as jnp
from jax import lax
from jax.experimental import pallas as pl
from jax.experimental.pallas import tpu as pltpu
from jax.experimental.pallas import tpu_sc as plsc

N_NODES = 10000
D = 128
N_CORES = 2
N_SUBCORES = 16
N_WORKERS = N_CORES * N_SUBCORES
CHUNK = 64
ACC_ROWS = 10240
FRAC0 = 0.60


def _matmul_kernel(x_ref, w_ref, o_ref):
    o_ref[...] = jnp.dot(x_ref[...], w_ref[...],
                         preferred_element_type=jnp.float32)


def _combine_kernel(p0_ref, p1_ref, o_ref):
    o_ref[...] = jnp.maximum(p0_ref[0] + p1_ref[0], 0.0)


def _sc_agg_body(x_hbm, pk_hbm, w_hbm, part_hbm,
                 pk_v, w_v, colbuf, rowbuf, rows2, acc_sh, sem_a, sem_b,
                 ssem_a, ssem_b, nc0, nc1):
    cid = lax.axis_index("c")
    sid = lax.axis_index("s")
    wid = cid * N_SUBCORES + sid
    n_chunks = jnp.where(cid == 0, nc0, nc1)
    rows_a = rows2.at[pl.ds(0, CHUNK)]
    rows_b = rows2.at[pl.ds(CHUNK, CHUNK)]

    pltpu.sync_copy(pk_hbm.at[wid], pk_v)
    pltpu.sync_copy(w_hbm.at[wid], w_v)

    def _unpack(ch, p):
        for g in range(CHUNK // 16):
            sl = pl.ds(g * 16, 16)
            v = pk_v[ch, sl]
            colbuf[p, sl] = v & 0xFFFF
            rowbuf[p, sl] = lax.shift_right_logical(v, 16)

    zeros16 = jnp.zeros((16,), jnp.float32)

    def _zero_row(i, _):
        for j in range(D // 16):
            rows2[i, pl.ds(j * 16, 16)] = zeros16
        return 0

    lax.fori_loop(0, 2 * CHUNK, _zero_row, 0)

    rows_per_sub = ACC_ROWS // N_SUBCORES
    for k in range(rows_per_sub // (2 * CHUNK)):
        pltpu.sync_copy(
            rows2,
            acc_sh.at[pl.ds(sid * rows_per_sub + k * 2 * CHUNK, 2 * CHUNK)])
    plsc.subcore_barrier()

    def _fire_gather(p, buf, sem):
        pltpu.async_copy(x_hbm.at[colbuf.at[p]], buf, sem)

    def _wait_gather(p, buf, sem):
        pltpu.make_async_copy(x_hbm.at[colbuf.at[p]], buf, sem).wait()

    def _wait_scatter(p, buf, ssem):
        pltpu.make_async_copy(buf, acc_sh.at[rowbuf.at[p]], ssem).wait()

    def _scale(ch, off):
        def _group_body(g, _):
            w16 = w_v[ch, pl.ds(g * 16, 16)]
            base = g * 16 + off
            for l in range(16):
                wvec = jnp.full((16,), w16[l], jnp.float32)
                for j in range(D // 16):
                    sl = pl.ds(j * 16, 16)
                    rows2[base + l, sl] = rows2[base + l, sl] * wvec
            return 0

        lax.fori_loop(0, CHUNK // 16, _group_body, 0)

    _unpack(0, 0)
    _fire_gather(0, rows_a, sem_a)

    def _pair_body(q, _):
        ch0 = 2 * q
        for b, (buf, off, sem, osem, obuf, ssem, ossem) in enumerate(
                ((rows_a, 0, sem_a, sem_b, rows_b, ssem_a, ssem_b),
                 (rows_b, CHUNK, sem_b, sem_a, rows_a, ssem_b, ssem_a))):
            ch = ch0 + b
            p, op = b, 1 - b
            if b == 0:
                @pl.when(ch0 >= 1)
                def _():
                    _wait_scatter(op, obuf, ossem)
            else:
                _wait_scatter(op, obuf, ossem)

            @pl.when(ch + 1 < n_chunks)
            def _():
                _unpack(ch + 1, op)
                _fire_gather(op, obuf, osem)

            _wait_gather(p, buf, sem)
            _scale(ch, off)
            pltpu.async_copy(buf, acc_sh.at[rowbuf.at[p]], ssem, add=True)
        return 0

    lax.fori_loop(0, n_chunks // 2, _pair_body, 0)
    _wait_scatter(1, rows_b, ssem_b)
    plsc.subcore_barrier()

    sl = pl.ds(sid * rows_per_sub, rows_per_sub)
    pltpu.sync_copy(acc_sh.at[sl], part_hbm.at[cid, sl])


def kernel(inputs, edge_index, edge_weight, W):
    n_edges = edge_index.shape[1]
    total = -(-n_edges // (N_SUBCORES * CHUNK))
    nc0 = 2 * int(round(FRAC0 * total / 2))
    nc1 = 2 * (-(-(total - nc0) // 2))
    ncmax = max(nc0, nc1)
    cap = N_SUBCORES * CHUNK * (nc0 + nc1)

    def _layout(a):
        a = jnp.pad(a, (0, cap - n_edges))
        s0 = a[:N_SUBCORES * nc0 * CHUNK].reshape(N_SUBCORES, nc0, CHUNK)
        s1 = a[N_SUBCORES * nc0 * CHUNK:].reshape(N_SUBCORES, nc1, CHUNK)
        s0 = jnp.pad(s0, ((0, 0), (0, ncmax - nc0), (0, 0)))
        s1 = jnp.pad(s1, ((0, 0), (0, ncmax - nc1), (0, 0)))
        return jnp.concatenate([s0, s1], axis=0)

    col = edge_index[1].astype(jnp.int32)
    row = edge_index[0].astype(jnp.int32)
    pk_p = _layout((row << 16) | col)
    w_p = _layout(edge_weight)

    n = inputs.shape[0]
    blk = 1000
    x = pl.pallas_call(
        _matmul_kernel,
        grid=(n // blk,),
        in_specs=[
            pl.BlockSpec((blk, D), lambda i: (i, 0)),
            pl.BlockSpec((D, D), lambda i: (0, 0)),
        ],
        out_specs=pl.BlockSpec((blk, D), lambda i: (i, 0)),
        out_shape=jax.ShapeDtypeStruct((n, D), jnp.float32),
    )(inputs, W)

    mesh = plsc.VectorSubcoreMesh(core_axis_name="c", subcore_axis_name="s")
    sc_agg = functools.partial(
        pl.kernel,
        out_type=jax.ShapeDtypeStruct((N_CORES, ACC_ROWS, D), jnp.float32),
        mesh=mesh,
        scratch_types=[
            pltpu.VMEM((ncmax, CHUNK), jnp.int32),
            pltpu.VMEM((ncmax, CHUNK), jnp.float32),
            pltpu.VMEM((2, CHUNK), jnp.int32),
            pltpu.VMEM((2, CHUNK), jnp.int32),
            pltpu.VMEM((2 * CHUNK, D), jnp.float32),
            pltpu.VMEM_SHARED((ACC_ROWS, D), jnp.float32),
            pltpu.SemaphoreType.DMA,
            pltpu.SemaphoreType.DMA,
            pltpu.SemaphoreType.DMA,
            pltpu.SemaphoreType.DMA,
        ],
        compiler_params=pltpu.CompilerParams(use_tc_tiling_on_sc=False),
    )(functools.partial(_sc_agg_body, nc0=nc0, nc1=nc1))
    part = sc_agg(x, pk_p, w_p)

    out = pl.pallas_call(
        _combine_kernel,
        grid=(n // blk,),
        in_specs=[
            pl.BlockSpec((1, blk, D), lambda i: (0, i, 0)),
            pl.BlockSpec((1, blk, D), lambda i: (1, i, 0)),
        ],
        out_specs=pl.BlockSpec((blk, D), lambda i: (i, 0)),
        out_shape=jax.ShapeDtypeStruct((n, D), jnp.float32),
    )(part, part)
    return out
